# baseline (device time: 22818 ns/iter reference)
import os

import jax
import jax.numpy as jnp
from jax import lax
from jax.experimental import pallas as pl
from jax.experimental.pallas import tpu as pltpu

_NOCOMM = os.environ.get("NOCOMM") == "1"

N_DEV = 8
B, SQ, D = 2, 128, 512
HQ, HKV, DH = 8, 2, 64
GROUP = HQ // HKV
SKV_SH = 128
NBLK = 2 * B
LANES = HKV * DH
SKV = N_DEV * SKV_SH

_SLOT_ORDER = (7, 6, 5, 4, 3, 2, 1)


def kernel(x, Wq, Wo, K_ext, V_ext):
    x2d = x.reshape(B * SQ, D)
    k2d = K_ext.reshape(B * SKV_SH, LANES)
    v2d = V_ext.reshape(B * SKV_SH, LANES)

    def body(x_ref, wq_ref, wo_ref, k_ref, v_ref, out_ref, kvfull,
             send_sems, recv_sems):
        my = lax.axis_index("i")

        barrier = pltpu.get_barrier_semaphore()
        for j in range(N_DEV - 1):
            peer = (my + 1 + j) % N_DEV
            pl.semaphore_signal(
                barrier, inc=1, device_id=(peer,),
                device_id_type=pl.DeviceIdType.MESH,
            )

        for b in range(B):
            kvfull[b * SKV:b * SKV + SKV_SH, :] = \
                k_ref[b * SKV_SH:(b + 1) * SKV_SH, :].astype(jnp.bfloat16)
            kvfull[(B + b) * SKV:(B + b) * SKV + SKV_SH, :] = \
                v_ref[b * SKV_SH:(b + 1) * SKV_SH, :].astype(jnp.bfloat16)

        pl.semaphore_wait(barrier, N_DEV - 1)

        sends = []
        for j in range(N_DEV - 1) if not _NOCOMM else []:
            peer = (my + 1 + j) % N_DEV
            slot = N_DEV - 1 - j
            for blk in range(NBLK):
                rdma = pltpu.make_async_remote_copy(
                    src_ref=kvfull.at[pl.ds(blk * SKV, SKV_SH)],
                    dst_ref=kvfull.at[pl.ds(blk * SKV + slot * SKV_SH, SKV_SH)],
                    send_sem=send_sems.at[j, blk],
                    recv_sem=recv_sems.at[slot, blk],
                    device_id=(peer,),
                    device_id_type=pl.DeviceIdType.MESH,
                )
                rdma.start()
                sends.append(rdma)

        q = jnp.dot(x_ref[:, :].astype(jnp.bfloat16),
                    wq_ref[:, :].astype(jnp.bfloat16),
                    preferred_element_type=jnp.float32)
        q = q * (0.125 * 1.4426950408889634)

        qgs, accs = [], []
        for b in range(B):
            for kh in range(HKV):
                qg = jnp.concatenate(
                    [q[b * SQ:(b + 1) * SQ,
                       (kh * GROUP + g) * DH:(kh * GROUP + g + 1) * DH]
                     for g in range(GROUP)], axis=0)
                qgs.append(qg.astype(jnp.bfloat16))
                accs.append(None)

        ones8 = jnp.ones((SKV_SH, 8), jnp.float32)

        def _fold(slot):
            for gi in range(B * HKV):
                b, kh = divmod(gi, HKV)
                lo, hi = kh * DH, (kh + 1) * DH
                kc = kvfull[b * SKV + slot * SKV_SH:
                            b * SKV + (slot + 1) * SKV_SH, lo:hi]
                vc = kvfull[(B + b) * SKV + slot * SKV_SH:
                            (B + b) * SKV + (slot + 1) * SKV_SH, lo:hi]
                s_mat = lax.dot_general(
                    qgs[gi], kc, (((1,), (1,)), ((), ())),
                    preferred_element_type=jnp.float32)
                p = jnp.exp2(s_mat)
                dl = lax.dot_general(
                    p, ones8, (((1,), (0,)), ((), ())),
                    preferred_element_type=jnp.float32)
                do = lax.dot_general(
                    p, vc.astype(jnp.float32), (((1,), (0,)), ((), ())),
                    preferred_element_type=jnp.float32)
                if accs[gi] is None:
                    accs[gi] = (dl, do)
                else:
                    l_acc, o_acc = accs[gi]
                    accs[gi] = (l_acc + dl, o_acc + do)

        _fold(0)

        for slot in _SLOT_ORDER if not _NOCOMM else ():
            for blk in range(NBLK):
                recv = pltpu.make_async_remote_copy(
                    src_ref=kvfull.at[pl.ds(blk * SKV, SKV_SH)],
                    dst_ref=kvfull.at[pl.ds(blk * SKV + slot * SKV_SH, SKV_SH)],
                    send_sem=send_sems.at[0, blk],
                    recv_sem=recv_sems.at[slot, blk],
                    device_id=(my,),
                    device_id_type=pl.DeviceIdType.MESH,
                )
                recv.wait_recv()
            _fold(slot)

        cats = []
        for b in range(B):
            head_outs = []
            for kh in range(HKV):
                l_acc, o_acc = accs[b * HKV + kh]
                oh = o_acc / l_acc[:, :1]
                head_outs.extend(
                    oh[g * SQ:(g + 1) * SQ, :] for g in range(GROUP))
            cats.append(jnp.concatenate(head_outs, axis=1))
        cat_all = jnp.concatenate(cats, axis=0)
        out_ref[:, :] = jnp.dot(cat_all.astype(jnp.bfloat16),
                                wo_ref[:, :].astype(jnp.bfloat16),
                                preferred_element_type=jnp.float32)

        for rdma in sends:
            rdma.wait_send()

    out2d = pl.pallas_call(
        body,
        out_shape=jax.ShapeDtypeStruct((B * SQ, D), jnp.float32),
        in_specs=[pl.BlockSpec(memory_space=pltpu.VMEM)] * 5,
        out_specs=pl.BlockSpec(memory_space=pltpu.VMEM),
        scratch_shapes=[
            pltpu.VMEM((NBLK * SKV, LANES), jnp.bfloat16),
            pltpu.SemaphoreType.DMA((N_DEV - 1, NBLK)),
            pltpu.SemaphoreType.DMA((N_DEV, NBLK)),
        ],
        compiler_params=pltpu.CompilerParams(collective_id=0),
    )(x2d, Wq, Wo, k2d, v2d)
    return out2d.reshape(B, SQ, D)


# device time: 22751 ns/iter; 1.0029x vs baseline; 1.0029x over previous
import os

import jax
import jax.numpy as jnp
from jax import lax
from jax.experimental import pallas as pl
from jax.experimental.pallas import tpu as pltpu

_NOCOMM = os.environ.get("NOCOMM") == "1"

N_DEV = 8
B, SQ, D = 2, 128, 512
HQ, HKV, DH = 8, 2, 64
GROUP = HQ // HKV
SKV_SH = 128
NBLK = 2 * B
LANES = HKV * DH
SKV = N_DEV * SKV_SH

_SLOT_ORDER = (7, 6, 5, 4, 3, 2, 1)


def kernel(x, Wq, Wo, K_ext, V_ext):
    x2d = x.reshape(B * SQ, D)
    k2d = K_ext.reshape(B * SKV_SH, LANES)
    v2d = V_ext.reshape(B * SKV_SH, LANES)

    def body(x_ref, wq_ref, wo_ref, k_ref, v_ref, out_ref, kvfull,
             send_sems, recv_sems):
        my = lax.axis_index("i")

        barrier = pltpu.get_barrier_semaphore()
        for j in range(N_DEV - 1):
            peer = (my + 1 + j) % N_DEV
            pl.semaphore_signal(
                barrier, inc=1, device_id=(peer,),
                device_id_type=pl.DeviceIdType.MESH,
            )

        for b in range(B):
            kvfull[b * SKV:b * SKV + SKV_SH, :] = \
                k_ref[b * SKV_SH:(b + 1) * SKV_SH, :].astype(jnp.bfloat16)
            kvfull[(B + b) * SKV:(B + b) * SKV + SKV_SH, :] = \
                v_ref[b * SKV_SH:(b + 1) * SKV_SH, :].astype(jnp.bfloat16)

        pl.semaphore_wait(barrier, N_DEV - 1)

        sends = []
        for j in range(N_DEV - 1) if not _NOCOMM else []:
            peer = (my + 1 + j) % N_DEV
            slot = N_DEV - 1 - j
            for blk in range(NBLK):
                rdma = pltpu.make_async_remote_copy(
                    src_ref=kvfull.at[pl.ds(blk * SKV, SKV_SH)],
                    dst_ref=kvfull.at[pl.ds(blk * SKV + slot * SKV_SH, SKV_SH)],
                    send_sem=send_sems.at[j, blk],
                    recv_sem=recv_sems.at[slot, blk],
                    device_id=(peer,),
                    device_id_type=pl.DeviceIdType.MESH,
                )
                rdma.start()
                sends.append(rdma)

        q = jnp.dot(x_ref[:, :].astype(jnp.bfloat16),
                    wq_ref[:, :].astype(jnp.bfloat16),
                    preferred_element_type=jnp.float32)
        q = q * (0.125 * 1.4426950408889634)

        qgs, accs = [], []
        for b in range(B):
            for kh in range(HKV):
                qg = jnp.concatenate(
                    [q[b * SQ:(b + 1) * SQ,
                       (kh * GROUP + g) * DH:(kh * GROUP + g + 1) * DH]
                     for g in range(GROUP)], axis=0)
                qgs.append(qg.astype(jnp.bfloat16))
                accs.append(None)

        ones8 = jnp.ones((SKV_SH, 8), jnp.bfloat16)

        def _fold(slot):
            for gi in range(B * HKV):
                b, kh = divmod(gi, HKV)
                lo, hi = kh * DH, (kh + 1) * DH
                kc = kvfull[b * SKV + slot * SKV_SH:
                            b * SKV + (slot + 1) * SKV_SH, lo:hi]
                vc = kvfull[(B + b) * SKV + slot * SKV_SH:
                            (B + b) * SKV + (slot + 1) * SKV_SH, lo:hi]
                s_mat = lax.dot_general(
                    qgs[gi], kc, (((1,), (1,)), ((), ())),
                    preferred_element_type=jnp.float32)
                p = jnp.exp2(s_mat).astype(jnp.bfloat16)
                dl = lax.dot_general(
                    p, ones8, (((1,), (0,)), ((), ())),
                    preferred_element_type=jnp.float32)
                do = lax.dot_general(
                    p, vc, (((1,), (0,)), ((), ())),
                    preferred_element_type=jnp.float32)
                if accs[gi] is None:
                    accs[gi] = (dl, do)
                else:
                    l_acc, o_acc = accs[gi]
                    accs[gi] = (l_acc + dl, o_acc + do)

        _fold(0)

        for slot in _SLOT_ORDER if not _NOCOMM else ():
            for blk in range(NBLK):
                recv = pltpu.make_async_remote_copy(
                    src_ref=kvfull.at[pl.ds(blk * SKV, SKV_SH)],
                    dst_ref=kvfull.at[pl.ds(blk * SKV + slot * SKV_SH, SKV_SH)],
                    send_sem=send_sems.at[0, blk],
                    recv_sem=recv_sems.at[slot, blk],
                    device_id=(my,),
                    device_id_type=pl.DeviceIdType.MESH,
                )
                recv.wait_recv()
            _fold(slot)

        cats = []
        for b in range(B):
            head_outs = []
            for kh in range(HKV):
                l_acc, o_acc = accs[b * HKV + kh]
                oh = o_acc / l_acc[:, :1]
                head_outs.extend(
                    oh[g * SQ:(g + 1) * SQ, :] for g in range(GROUP))
            cats.append(jnp.concatenate(head_outs, axis=1))
        cat_all = jnp.concatenate(cats, axis=0)
        out_ref[:, :] = jnp.dot(cat_all.astype(jnp.bfloat16),
                                wo_ref[:, :].astype(jnp.bfloat16),
                                preferred_element_type=jnp.float32)

        for rdma in sends:
            rdma.wait_send()

    out2d = pl.pallas_call(
        body,
        out_shape=jax.ShapeDtypeStruct((B * SQ, D), jnp.float32),
        in_specs=[pl.BlockSpec(memory_space=pltpu.VMEM)] * 5,
        out_specs=pl.BlockSpec(memory_space=pltpu.VMEM),
        scratch_shapes=[
            pltpu.VMEM((NBLK * SKV, LANES), jnp.bfloat16),
            pltpu.SemaphoreType.DMA((N_DEV - 1, NBLK)),
            pltpu.SemaphoreType.DMA((N_DEV, NBLK)),
        ],
        compiler_params=pltpu.CompilerParams(collective_id=0),
    )(x2d, Wq, Wo, k2d, v2d)
    return out2d.reshape(B, SQ, D)


# device time: 20648 ns/iter; 1.1051x vs baseline; 1.1019x over previous
import os

import jax
import jax.numpy as jnp
from jax import lax
from jax.experimental import pallas as pl
from jax.experimental.pallas import tpu as pltpu

_NOCOMM = os.environ.get("NOCOMM") == "1"

N_DEV = 8
B, SQ, D = 2, 128, 512
HQ, HKV, DH = 8, 2, 64
GROUP = HQ // HKV
SKV_SH = 128
NBLK = 2 * B
LANES = HKV * DH
SKV = N_DEV * SKV_SH

_SLOT_ORDER = (7, 6, 5, 4, 3, 2, 1)


def kernel(x, Wq, Wo, K_ext, V_ext):
    x2d = x.reshape(B * SQ, D)
    k2d = K_ext.reshape(B * SKV_SH, LANES)
    v2d = V_ext.reshape(B * SKV_SH, LANES)

    def body(x_ref, wq_ref, wo_ref, k_ref, v_ref, out_ref, kvfull,
             send_sems, recv_sems):
        my = lax.axis_index("i")

        barrier = pltpu.get_barrier_semaphore()
        for j in range(N_DEV - 1):
            peer = (my + 1 + j) % N_DEV
            pl.semaphore_signal(
                barrier, inc=1, device_id=(peer,),
                device_id_type=pl.DeviceIdType.MESH,
            )

        for b in range(B):
            kvfull[b * SKV:b * SKV + SKV_SH, :] = \
                k_ref[b * SKV_SH:(b + 1) * SKV_SH, :].astype(jnp.bfloat16)
            kvfull[(B + b) * SKV:(B + b) * SKV + SKV_SH, :] = \
                v_ref[b * SKV_SH:(b + 1) * SKV_SH, :].astype(jnp.bfloat16)

        pl.semaphore_wait(barrier, N_DEV - 1)

        sends = []
        for j in range(N_DEV - 1) if not _NOCOMM else []:
            peer = (my + 1 + j) % N_DEV
            slot = N_DEV - 1 - j
            for blk in range(NBLK):
                rdma = pltpu.make_async_remote_copy(
                    src_ref=kvfull.at[pl.ds(blk * SKV, SKV_SH)],
                    dst_ref=kvfull.at[pl.ds(blk * SKV + slot * SKV_SH, SKV_SH)],
                    send_sem=send_sems.at[j, blk],
                    recv_sem=recv_sems.at[slot, blk],
                    device_id=(peer,),
                    device_id_type=pl.DeviceIdType.MESH,
                )
                rdma.start()
                sends.append(rdma)

        q = jnp.dot(x_ref[:, :].astype(jnp.bfloat16),
                    wq_ref[:, :].astype(jnp.bfloat16),
                    preferred_element_type=jnp.float32)
        q = q * (0.125 * 1.4426950408889634)

        qgs, accs = [], []
        for b in range(B):
            for kh in range(HKV):
                qg = jnp.concatenate(
                    [q[b * SQ:(b + 1) * SQ,
                       (kh * GROUP + g) * DH:(kh * GROUP + g + 1) * DH]
                     for g in range(GROUP)], axis=0)
                qgs.append(qg.astype(jnp.bfloat16))
                accs.append(None)

        def _fold(slot):
            for gi in range(B * HKV):
                b, kh = divmod(gi, HKV)
                lo, hi = kh * DH, (kh + 1) * DH
                kc = kvfull[b * SKV + slot * SKV_SH:
                            b * SKV + (slot + 1) * SKV_SH, lo:hi]
                vc = kvfull[(B + b) * SKV + slot * SKV_SH:
                            (B + b) * SKV + (slot + 1) * SKV_SH, lo:hi]
                s_mat = lax.dot_general(
                    qgs[gi], kc, (((1,), (1,)), ((), ())),
                    preferred_element_type=jnp.float32)
                p = jnp.exp2(s_mat)
                dl = jnp.sum(p, axis=1, keepdims=True)
                do = lax.dot_general(
                    p.astype(jnp.bfloat16), vc, (((1,), (0,)), ((), ())),
                    preferred_element_type=jnp.float32)
                if accs[gi] is None:
                    accs[gi] = (dl, do)
                else:
                    l_acc, o_acc = accs[gi]
                    accs[gi] = (l_acc + dl, o_acc + do)

        _fold(0)

        for slot in _SLOT_ORDER if not _NOCOMM else ():
            for blk in range(NBLK):
                recv = pltpu.make_async_remote_copy(
                    src_ref=kvfull.at[pl.ds(blk * SKV, SKV_SH)],
                    dst_ref=kvfull.at[pl.ds(blk * SKV + slot * SKV_SH, SKV_SH)],
                    send_sem=send_sems.at[0, blk],
                    recv_sem=recv_sems.at[slot, blk],
                    device_id=(my,),
                    device_id_type=pl.DeviceIdType.MESH,
                )
                recv.wait_recv()
            _fold(slot)

        cats = []
        for b in range(B):
            head_outs = []
            for kh in range(HKV):
                l_acc, o_acc = accs[b * HKV + kh]
                oh = o_acc / l_acc
                head_outs.extend(
                    oh[g * SQ:(g + 1) * SQ, :] for g in range(GROUP))
            cats.append(jnp.concatenate(head_outs, axis=1))
        cat_all = jnp.concatenate(cats, axis=0)
        out_ref[:, :] = jnp.dot(cat_all.astype(jnp.bfloat16),
                                wo_ref[:, :].astype(jnp.bfloat16),
                                preferred_element_type=jnp.float32)

        for rdma in sends:
            rdma.wait_send()

    out2d = pl.pallas_call(
        body,
        out_shape=jax.ShapeDtypeStruct((B * SQ, D), jnp.float32),
        in_specs=[pl.BlockSpec(memory_space=pltpu.VMEM)] * 5,
        out_specs=pl.BlockSpec(memory_space=pltpu.VMEM),
        scratch_shapes=[
            pltpu.VMEM((NBLK * SKV, LANES), jnp.bfloat16),
            pltpu.SemaphoreType.DMA((N_DEV - 1, NBLK)),
            pltpu.SemaphoreType.DMA((N_DEV, NBLK)),
        ],
        compiler_params=pltpu.CompilerParams(collective_id=0),
    )(x2d, Wq, Wo, k2d, v2d)
    return out2d.reshape(B, SQ, D)


# device time: 20438 ns/iter; 1.1164x vs baseline; 1.0103x over previous
import os

import jax
import jax.numpy as jnp
from jax import lax
from jax.experimental import pallas as pl
from jax.experimental.pallas import tpu as pltpu

_NOCOMM = os.environ.get("NOCOMM") == "1"

N_DEV = 8
B, SQ, D = 2, 128, 512
HQ, HKV, DH = 8, 2, 64
GROUP = HQ // HKV
SKV_SH = 128
NBLK = 2 * B
LANES = HKV * DH
SKV = N_DEV * SKV_SH

_SLOT_ORDER = (7, 6, 5, 4, 3, 2, 1)


def kernel(x, Wq, Wo, K_ext, V_ext):
    x2d = x.reshape(B * SQ, D)
    k2d = K_ext.reshape(B * SKV_SH, LANES)
    v2d = V_ext.reshape(B * SKV_SH, LANES)

    def body(x_ref, wq_ref, wo_ref, k_ref, v_ref, out_ref, kvfull,
             send_sems, recv_sems):
        my = lax.axis_index("i")

        barrier = pltpu.get_barrier_semaphore()
        for j in range(N_DEV - 1):
            peer = (my + 1 + j) % N_DEV
            pl.semaphore_signal(
                barrier, inc=1, device_id=(peer,),
                device_id_type=pl.DeviceIdType.MESH,
            )

        for b in range(B):
            kvfull[b * SKV:b * SKV + SKV_SH, :] = \
                k_ref[b * SKV_SH:(b + 1) * SKV_SH, :].astype(jnp.bfloat16)
            kvfull[(B + b) * SKV:(B + b) * SKV + SKV_SH, :] = \
                v_ref[b * SKV_SH:(b + 1) * SKV_SH, :].astype(jnp.bfloat16)

        pl.semaphore_wait(barrier, N_DEV - 1)

        sends = []
        for j in range(N_DEV - 1) if not _NOCOMM else []:
            peer = (my + 1 + j) % N_DEV
            slot = N_DEV - 1 - j
            for blk in range(NBLK):
                rdma = pltpu.make_async_remote_copy(
                    src_ref=kvfull.at[pl.ds(blk * SKV, SKV_SH)],
                    dst_ref=kvfull.at[pl.ds(blk * SKV + slot * SKV_SH, SKV_SH)],
                    send_sem=send_sems.at[j, blk],
                    recv_sem=recv_sems.at[slot, blk],
                    device_id=(peer,),
                    device_id_type=pl.DeviceIdType.MESH,
                )
                rdma.start()
                sends.append(rdma)

        q = jnp.dot(x_ref[:, :].astype(jnp.bfloat16),
                    wq_ref[:, :].astype(jnp.bfloat16),
                    preferred_element_type=jnp.float32)
        q = q * (0.125 * 1.4426950408889634)

        qgs, accs = [], []
        for b in range(B):
            for kh in range(HKV):
                qg = jnp.concatenate(
                    [q[b * SQ:(b + 1) * SQ,
                       (kh * GROUP + g) * DH:(kh * GROUP + g + 1) * DH]
                     for g in range(GROUP)], axis=0)
                qgs.append(qg.astype(jnp.bfloat16))
                accs.append(None)

        def _fold(slot, nsl=1):
            for gi in range(B * HKV):
                b, kh = divmod(gi, HKV)
                lo, hi = kh * DH, (kh + 1) * DH
                kc = kvfull[b * SKV + slot * SKV_SH:
                            b * SKV + (slot + nsl) * SKV_SH, lo:hi]
                vc = kvfull[(B + b) * SKV + slot * SKV_SH:
                            (B + b) * SKV + (slot + nsl) * SKV_SH, lo:hi]
                s_mat = lax.dot_general(
                    qgs[gi], kc, (((1,), (1,)), ((), ())),
                    preferred_element_type=jnp.float32)
                p = jnp.exp2(s_mat)
                dl = jnp.sum(p, axis=1, keepdims=True)
                do = lax.dot_general(
                    p.astype(jnp.bfloat16), vc, (((1,), (0,)), ((), ())),
                    preferred_element_type=jnp.float32)
                if accs[gi] is None:
                    accs[gi] = (dl, do)
                else:
                    l_acc, o_acc = accs[gi]
                    accs[gi] = (l_acc + dl, o_acc + do)

        _fold(0)

        def _wait_slot(slot):
            for blk in range(NBLK):
                recv = pltpu.make_async_remote_copy(
                    src_ref=kvfull.at[pl.ds(blk * SKV, SKV_SH)],
                    dst_ref=kvfull.at[pl.ds(blk * SKV + slot * SKV_SH, SKV_SH)],
                    send_sem=send_sems.at[0, blk],
                    recv_sem=recv_sems.at[slot, blk],
                    device_id=(my,),
                    device_id_type=pl.DeviceIdType.MESH,
                )
                recv.wait_recv()

        if not _NOCOMM:
            for hi_slot in (7, 5, 3):
                _wait_slot(hi_slot)
                _wait_slot(hi_slot - 1)
                _fold(hi_slot - 1, nsl=2)
            _wait_slot(1)
            _fold(1)
        else:
            for hi_slot in (7, 5, 3):
                _fold(hi_slot - 1, nsl=2)
            _fold(1)

        cats = []
        for b in range(B):
            head_outs = []
            for kh in range(HKV):
                l_acc, o_acc = accs[b * HKV + kh]
                oh = o_acc / l_acc
                head_outs.extend(
                    oh[g * SQ:(g + 1) * SQ, :] for g in range(GROUP))
            cats.append(jnp.concatenate(head_outs, axis=1))
        cat_all = jnp.concatenate(cats, axis=0)
        out_ref[:, :] = jnp.dot(cat_all.astype(jnp.bfloat16),
                                wo_ref[:, :].astype(jnp.bfloat16),
                                preferred_element_type=jnp.float32)

        for rdma in sends:
            rdma.wait_send()

    out2d = pl.pallas_call(
        body,
        out_shape=jax.ShapeDtypeStruct((B * SQ, D), jnp.float32),
        in_specs=[pl.BlockSpec(memory_space=pltpu.VMEM)] * 5,
        out_specs=pl.BlockSpec(memory_space=pltpu.VMEM),
        scratch_shapes=[
            pltpu.VMEM((NBLK * SKV, LANES), jnp.bfloat16),
            pltpu.SemaphoreType.DMA((N_DEV - 1, NBLK)),
            pltpu.SemaphoreType.DMA((N_DEV, NBLK)),
        ],
        compiler_params=pltpu.CompilerParams(collective_id=0),
    )(x2d, Wq, Wo, k2d, v2d)
    return out2d.reshape(B, SQ, D)


# device time: 17842 ns/iter; 1.2789x vs baseline; 1.1455x over previous
import os

import jax
import jax.numpy as jnp
from jax import lax
from jax.experimental import pallas as pl
from jax.experimental.pallas import tpu as pltpu

_NOCOMM = os.environ.get("NOCOMM") == "1"

N_DEV = 8
B, SQ, D = 2, 128, 512
HQ, HKV, DH = 8, 2, 64
GROUP = HQ // HKV
SKV_SH = 128
LANES = HKV * DH
QR = B * SQ
SLICE = QR // N_DEV
PW = HQ * DH + LANES


def kernel(x, Wq, Wo, K_ext, V_ext):
    x2d = x.reshape(QR, D)
    k2d = K_ext.reshape(SKV_SH * B, LANES)
    v2d = V_ext.reshape(SKV_SH * B, LANES)

    def body(x_ref, wq_ref, wo_ref, k_ref, v_ref, out_ref,
             part_s, pbuf, ogather, s1_sems, r1_sems, s2_sems, r2_sems):
        my = lax.axis_index("i")

        barrier = pltpu.get_barrier_semaphore()
        for j in range(N_DEV - 1):
            peer = (my + 1 + j) % N_DEV
            pl.semaphore_signal(
                barrier, inc=1, device_id=(peer,),
                device_id_type=pl.DeviceIdType.MESH,
            )

        q = jnp.dot(x_ref[:, :].astype(jnp.bfloat16),
                    wq_ref[:, :].astype(jnp.bfloat16),
                    preferred_element_type=jnp.float32)
        q = q * (0.125 * 1.4426950408889634)

        o_parts, l_parts = [], []
        for b in range(B):
            for kh in range(HKV):
                lo, hi = kh * DH, (kh + 1) * DH
                qg = jnp.concatenate(
                    [q[b * SQ:(b + 1) * SQ,
                       (kh * GROUP + g) * DH:(kh * GROUP + g + 1) * DH]
                     for g in range(GROUP)], axis=0).astype(jnp.bfloat16)
                kc = k_ref[b * SKV_SH:(b + 1) * SKV_SH, lo:hi].astype(jnp.bfloat16)
                vc = v_ref[b * SKV_SH:(b + 1) * SKV_SH, lo:hi].astype(jnp.bfloat16)
                s_mat = lax.dot_general(
                    qg, kc, (((1,), (1,)), ((), ())),
                    preferred_element_type=jnp.float32)
                p = jnp.exp2(s_mat)
                l_parts.append(jnp.sum(p, axis=1, keepdims=True))
                o_parts.append(lax.dot_general(
                    p.astype(jnp.bfloat16), vc, (((1,), (0,)), ((), ())),
                    preferred_element_type=jnp.float32))

        rows = []
        for b in range(B):
            ob = jnp.concatenate(
                [o_parts[b * HKV + kh][g * SQ:(g + 1) * SQ, :]
                 for kh in range(HKV) for g in range(GROUP)], axis=1)
            lb = jnp.concatenate(
                [l_parts[b * HKV + kh][g * SQ:(g + 1) * SQ, :]
                 for kh in range(HKV) for g in range(GROUP)]
                + [jnp.zeros((SQ, LANES - HQ), jnp.float32)], axis=1)
            rows.append(jnp.concatenate([ob, lb], axis=1))
        part_s[:, :] = jnp.concatenate(rows, axis=0).astype(jnp.bfloat16)

        pbuf[pl.ds(my * SLICE, SLICE), :] = part_s[pl.ds(my * SLICE, SLICE), :]

        pl.semaphore_wait(barrier, N_DEV - 1)

        sends = []
        for j in range(N_DEV - 1) if not _NOCOMM else []:
            peer = (my + 1 + j) % N_DEV
            rdma = pltpu.make_async_remote_copy(
                src_ref=part_s.at[pl.ds(peer * SLICE, SLICE)],
                dst_ref=pbuf.at[pl.ds(my * SLICE, SLICE)],
                send_sem=s1_sems.at[j],
                recv_sem=r1_sems.at[my],
                device_id=(peer,),
                device_id_type=pl.DeviceIdType.MESH,
            )
            rdma.start()
            sends.append(rdma)

        for j in range(N_DEV - 1) if not _NOCOMM else []:
            src = (my + 1 + j) % N_DEV
            recv = pltpu.make_async_remote_copy(
                src_ref=part_s.at[pl.ds(src * SLICE, SLICE)],
                dst_ref=pbuf.at[pl.ds(src * SLICE, SLICE)],
                send_sem=s1_sems.at[j],
                recv_sem=r1_sems.at[src],
                device_id=(my,),
                device_id_type=pl.DeviceIdType.MESH,
            )
            recv.wait_recv()

        acc = pbuf[0 * SLICE:1 * SLICE, :].astype(jnp.float32)
        for s in range(1, N_DEV):
            acc = acc + pbuf[s * SLICE:(s + 1) * SLICE, :].astype(jnp.float32)
        oh = jnp.concatenate(
            [acc[:, h * DH:(h + 1) * DH] / acc[:, HQ * DH + h:HQ * DH + h + 1]
             for h in range(HQ)], axis=1)
        out_slice = jnp.dot(oh.astype(jnp.bfloat16),
                            wo_ref[:, :].astype(jnp.bfloat16),
                            preferred_element_type=jnp.float32)
        ogather[pl.ds(my * SLICE, SLICE), :] = out_slice.astype(jnp.bfloat16)

        for j in range(N_DEV - 1) if not _NOCOMM else []:
            peer = (my + 1 + j) % N_DEV
            rdma = pltpu.make_async_remote_copy(
                src_ref=ogather.at[pl.ds(my * SLICE, SLICE)],
                dst_ref=ogather.at[pl.ds(my * SLICE, SLICE)],
                send_sem=s2_sems.at[j],
                recv_sem=r2_sems.at[my],
                device_id=(peer,),
                device_id_type=pl.DeviceIdType.MESH,
            )
            rdma.start()
            sends.append(rdma)

        for j in range(N_DEV - 1) if not _NOCOMM else []:
            src = (my + 1 + j) % N_DEV
            recv = pltpu.make_async_remote_copy(
                src_ref=ogather.at[pl.ds(src * SLICE, SLICE)],
                dst_ref=ogather.at[pl.ds(src * SLICE, SLICE)],
                send_sem=s2_sems.at[j],
                recv_sem=r2_sems.at[src],
                device_id=(my,),
                device_id_type=pl.DeviceIdType.MESH,
            )
            recv.wait_recv()

        out_ref[:, :] = ogather[:, :].astype(jnp.float32)

        for rdma in sends:
            rdma.wait_send()

    out2d = pl.pallas_call(
        body,
        out_shape=jax.ShapeDtypeStruct((QR, D), jnp.float32),
        in_specs=[pl.BlockSpec(memory_space=pltpu.VMEM)] * 5,
        out_specs=pl.BlockSpec(memory_space=pltpu.VMEM),
        scratch_shapes=[
            pltpu.VMEM((QR, PW), jnp.bfloat16),
            pltpu.VMEM((QR, PW), jnp.bfloat16),
            pltpu.VMEM((QR, D), jnp.bfloat16),
            pltpu.SemaphoreType.DMA((N_DEV - 1,)),
            pltpu.SemaphoreType.DMA((N_DEV,)),
            pltpu.SemaphoreType.DMA((N_DEV - 1,)),
            pltpu.SemaphoreType.DMA((N_DEV,)),
        ],
        compiler_params=pltpu.CompilerParams(collective_id=0),
    )(x2d, Wq, Wo, k2d, v2d)
    return out2d.reshape(B, SQ, D)


# device time: 17830 ns/iter; 1.2798x vs baseline; 1.0007x over previous
import os

import jax
import jax.numpy as jnp
from jax import lax
from jax.experimental import pallas as pl
from jax.experimental.pallas import tpu as pltpu

_NOCOMM = os.environ.get("NOCOMM") == "1"

N_DEV = 8
B, SQ, D = 2, 128, 512
HQ, HKV, DH = 8, 2, 64
GROUP = HQ // HKV
SKV_SH = 128
LANES = HKV * DH
QR = B * SQ
SLICE = QR // N_DEV
PW = HQ * DH + LANES


def kernel(x, Wq, Wo, K_ext, V_ext):
    x2d = x.reshape(QR, D)
    k2d = K_ext.reshape(SKV_SH * B, LANES)
    v2d = V_ext.reshape(SKV_SH * B, LANES)

    def body(x_ref, wq_ref, wo_ref, k_ref, v_ref, out_ref,
             part_s, pbuf, ogather, s1_sems, r1_sems, s2_sems, r2_sems):
        my = lax.axis_index("i")

        barrier = pltpu.get_barrier_semaphore()
        for j in range(N_DEV - 1):
            peer = (my + 1 + j) % N_DEV
            pl.semaphore_signal(
                barrier, inc=1, device_id=(peer,),
                device_id_type=pl.DeviceIdType.MESH,
            )

        q = jnp.dot(x_ref[:, :].astype(jnp.bfloat16),
                    wq_ref[:, :].astype(jnp.bfloat16),
                    preferred_element_type=jnp.float32)
        q = q * (0.125 * 1.4426950408889634)

        o_parts, l_parts = [], []
        for b in range(B):
            for kh in range(HKV):
                lo, hi = kh * DH, (kh + 1) * DH
                qg = jnp.concatenate(
                    [q[b * SQ:(b + 1) * SQ,
                       (kh * GROUP + g) * DH:(kh * GROUP + g + 1) * DH]
                     for g in range(GROUP)], axis=0).astype(jnp.bfloat16)
                kc = k_ref[b * SKV_SH:(b + 1) * SKV_SH, lo:hi].astype(jnp.bfloat16)
                vc = v_ref[b * SKV_SH:(b + 1) * SKV_SH, lo:hi].astype(jnp.bfloat16)
                s_mat = lax.dot_general(
                    qg, kc, (((1,), (1,)), ((), ())),
                    preferred_element_type=jnp.float32)
                p = jnp.exp2(s_mat)
                l_parts.append(jnp.sum(p, axis=1, keepdims=True))
                o_parts.append(lax.dot_general(
                    p.astype(jnp.bfloat16), vc, (((1,), (0,)), ((), ())),
                    preferred_element_type=jnp.float32))

        rows = []
        for b in range(B):
            ob = jnp.concatenate(
                [o_parts[b * HKV + kh][g * SQ:(g + 1) * SQ, :]
                 for kh in range(HKV) for g in range(GROUP)], axis=1)
            lb = jnp.concatenate(
                [l_parts[b * HKV + kh][g * SQ:(g + 1) * SQ, :]
                 for kh in range(HKV) for g in range(GROUP)]
                + [jnp.zeros((SQ, LANES - HQ), jnp.float32)], axis=1)
            rows.append(jnp.concatenate([ob, lb], axis=1))
        part_s[:, :] = jnp.concatenate(rows, axis=0).astype(jnp.bfloat16)

        pbuf[pl.ds(my * SLICE, SLICE), :] = part_s[pl.ds(my * SLICE, SLICE), :]

        pl.semaphore_wait(barrier, N_DEV - 1)

        sends = []
        for j in range(N_DEV - 1) if not _NOCOMM else []:
            peer = (my + 1 + j) % N_DEV
            rdma = pltpu.make_async_remote_copy(
                src_ref=part_s.at[pl.ds(peer * SLICE, SLICE)],
                dst_ref=pbuf.at[pl.ds(my * SLICE, SLICE)],
                send_sem=s1_sems.at[j],
                recv_sem=r1_sems.at[my],
                device_id=(peer,),
                device_id_type=pl.DeviceIdType.MESH,
            )
            rdma.start()
            sends.append(rdma)

        acc = pbuf[pl.ds(my * SLICE, SLICE), :].astype(jnp.float32)
        for j in range(N_DEV - 1) if not _NOCOMM else []:
            src = (my - 1 - j) % N_DEV
            recv = pltpu.make_async_remote_copy(
                src_ref=part_s.at[pl.ds(src * SLICE, SLICE)],
                dst_ref=pbuf.at[pl.ds(src * SLICE, SLICE)],
                send_sem=s1_sems.at[j],
                recv_sem=r1_sems.at[src],
                device_id=(my,),
                device_id_type=pl.DeviceIdType.MESH,
            )
            recv.wait_recv()
            acc = acc + pbuf[pl.ds(src * SLICE, SLICE), :].astype(jnp.float32)
        oh = jnp.concatenate(
            [acc[:, h * DH:(h + 1) * DH] / acc[:, HQ * DH + h:HQ * DH + h + 1]
             for h in range(HQ)], axis=1)
        out_slice = jnp.dot(oh.astype(jnp.bfloat16),
                            wo_ref[:, :].astype(jnp.bfloat16),
                            preferred_element_type=jnp.float32)
        ogather[pl.ds(my * SLICE, SLICE), :] = out_slice.astype(jnp.bfloat16)

        for j in range(N_DEV - 1) if not _NOCOMM else []:
            peer = (my + 1 + j) % N_DEV
            rdma = pltpu.make_async_remote_copy(
                src_ref=ogather.at[pl.ds(my * SLICE, SLICE)],
                dst_ref=ogather.at[pl.ds(my * SLICE, SLICE)],
                send_sem=s2_sems.at[j],
                recv_sem=r2_sems.at[my],
                device_id=(peer,),
                device_id_type=pl.DeviceIdType.MESH,
            )
            rdma.start()
            sends.append(rdma)

        for j in range(N_DEV - 1) if not _NOCOMM else []:
            src = (my - 1 - j) % N_DEV
            recv = pltpu.make_async_remote_copy(
                src_ref=ogather.at[pl.ds(src * SLICE, SLICE)],
                dst_ref=ogather.at[pl.ds(src * SLICE, SLICE)],
                send_sem=s2_sems.at[j],
                recv_sem=r2_sems.at[src],
                device_id=(my,),
                device_id_type=pl.DeviceIdType.MESH,
            )
            recv.wait_recv()

        out_ref[:, :] = ogather[:, :].astype(jnp.float32)

        for rdma in sends:
            rdma.wait_send()

    out2d = pl.pallas_call(
        body,
        out_shape=jax.ShapeDtypeStruct((QR, D), jnp.float32),
        in_specs=[pl.BlockSpec(memory_space=pltpu.VMEM)] * 5,
        out_specs=pl.BlockSpec(memory_space=pltpu.VMEM),
        scratch_shapes=[
            pltpu.VMEM((QR, PW), jnp.bfloat16),
            pltpu.VMEM((QR, PW), jnp.bfloat16),
            pltpu.VMEM((QR, D), jnp.bfloat16),
            pltpu.SemaphoreType.DMA((N_DEV - 1,)),
            pltpu.SemaphoreType.DMA((N_DEV,)),
            pltpu.SemaphoreType.DMA((N_DEV - 1,)),
            pltpu.SemaphoreType.DMA((N_DEV,)),
        ],
        compiler_params=pltpu.CompilerParams(collective_id=0),
    )(x2d, Wq, Wo, k2d, v2d)
    return out2d.reshape(B, SQ, D)


# device time: 17706 ns/iter; 1.2887x vs baseline; 1.0070x over previous
import os

import jax
import jax.numpy as jnp
from jax import lax
from jax.experimental import pallas as pl
from jax.experimental.pallas import tpu as pltpu

_NOCOMM = os.environ.get("NOCOMM") == "1"

N_DEV = 8
B, SQ, D = 2, 128, 512
HQ, HKV, DH = 8, 2, 64
GROUP = HQ // HKV
SKV_SH = 128
LANES = HKV * DH
QR = B * SQ
SLICE = QR // N_DEV
PW = HQ * DH + LANES


def kernel(x, Wq, Wo, K_ext, V_ext):
    def body(x_ref, wq_ref, wo_ref, k_ref, v_ref, out_ref,
             part_s, pbuf, ogather, s1_sems, r1_sems, s2_sems, r2_sems):
        my = lax.axis_index("i")

        barrier = pltpu.get_barrier_semaphore()
        for j in range(N_DEV - 1):
            peer = (my + 1 + j) % N_DEV
            pl.semaphore_signal(
                barrier, inc=1, device_id=(peer,),
                device_id_type=pl.DeviceIdType.MESH,
            )

        wq_bf = wq_ref[:, :].astype(jnp.bfloat16)
        qs = []
        for b in range(B):
            qb = jnp.dot(x_ref[b].astype(jnp.bfloat16), wq_bf,
                         preferred_element_type=jnp.float32)
            qs.append(qb * (0.125 * 1.4426950408889634))

        o_parts, l_parts = [], []
        for b in range(B):
            for kh in range(HKV):
                qg = jnp.concatenate(
                    [qs[b][:, (kh * GROUP + g) * DH:(kh * GROUP + g + 1) * DH]
                     for g in range(GROUP)], axis=0).astype(jnp.bfloat16)
                kc = k_ref[b, :, kh, :].astype(jnp.bfloat16)
                vc = v_ref[b, :, kh, :].astype(jnp.bfloat16)
                s_mat = lax.dot_general(
                    qg, kc, (((1,), (1,)), ((), ())),
                    preferred_element_type=jnp.float32)
                p = jnp.exp2(s_mat)
                l_parts.append(jnp.sum(p, axis=1, keepdims=True))
                o_parts.append(lax.dot_general(
                    p.astype(jnp.bfloat16), vc, (((1,), (0,)), ((), ())),
                    preferred_element_type=jnp.float32))

        rows = []
        for b in range(B):
            ob = jnp.concatenate(
                [o_parts[b * HKV + kh][g * SQ:(g + 1) * SQ, :]
                 for kh in range(HKV) for g in range(GROUP)], axis=1)
            lb = jnp.concatenate(
                [l_parts[b * HKV + kh][g * SQ:(g + 1) * SQ, :]
                 for kh in range(HKV) for g in range(GROUP)]
                + [jnp.zeros((SQ, LANES - HQ), jnp.float32)], axis=1)
            rows.append(jnp.concatenate([ob, lb], axis=1))
        part_s[:, :] = jnp.concatenate(rows, axis=0).astype(jnp.bfloat16)

        pbuf[pl.ds(my * SLICE, SLICE), :] = part_s[pl.ds(my * SLICE, SLICE), :]

        pl.semaphore_wait(barrier, N_DEV - 1)

        sends = []
        for j in range(N_DEV - 1) if not _NOCOMM else []:
            peer = (my + 1 + j) % N_DEV
            rdma = pltpu.make_async_remote_copy(
                src_ref=part_s.at[pl.ds(peer * SLICE, SLICE)],
                dst_ref=pbuf.at[pl.ds(my * SLICE, SLICE)],
                send_sem=s1_sems.at[j],
                recv_sem=r1_sems.at[my],
                device_id=(peer,),
                device_id_type=pl.DeviceIdType.MESH,
            )
            rdma.start()
            sends.append(rdma)

        acc = pbuf[pl.ds(my * SLICE, SLICE), :].astype(jnp.float32)
        for j in range(N_DEV - 1) if not _NOCOMM else []:
            src = (my - 1 - j) % N_DEV
            recv = pltpu.make_async_remote_copy(
                src_ref=part_s.at[pl.ds(src * SLICE, SLICE)],
                dst_ref=pbuf.at[pl.ds(src * SLICE, SLICE)],
                send_sem=s1_sems.at[j],
                recv_sem=r1_sems.at[src],
                device_id=(my,),
                device_id_type=pl.DeviceIdType.MESH,
            )
            recv.wait_recv()
            acc = acc + pbuf[pl.ds(src * SLICE, SLICE), :].astype(jnp.float32)
        oh = jnp.concatenate(
            [acc[:, h * DH:(h + 1) * DH] / acc[:, HQ * DH + h:HQ * DH + h + 1]
             for h in range(HQ)], axis=1)
        out_slice = jnp.dot(oh.astype(jnp.bfloat16),
                            wo_ref[:, :].astype(jnp.bfloat16),
                            preferred_element_type=jnp.float32)
        ogather[pl.ds(my * SLICE, SLICE), :] = out_slice.astype(jnp.bfloat16)

        for j in range(N_DEV - 1) if not _NOCOMM else []:
            peer = (my + 1 + j) % N_DEV
            rdma = pltpu.make_async_remote_copy(
                src_ref=ogather.at[pl.ds(my * SLICE, SLICE)],
                dst_ref=ogather.at[pl.ds(my * SLICE, SLICE)],
                send_sem=s2_sems.at[j],
                recv_sem=r2_sems.at[my],
                device_id=(peer,),
                device_id_type=pl.DeviceIdType.MESH,
            )
            rdma.start()
            sends.append(rdma)

        for j in range(N_DEV - 1) if not _NOCOMM else []:
            src = (my - 1 - j) % N_DEV
            recv = pltpu.make_async_remote_copy(
                src_ref=ogather.at[pl.ds(src * SLICE, SLICE)],
                dst_ref=ogather.at[pl.ds(src * SLICE, SLICE)],
                send_sem=s2_sems.at[j],
                recv_sem=r2_sems.at[src],
                device_id=(my,),
                device_id_type=pl.DeviceIdType.MESH,
            )
            recv.wait_recv()

        out_ref[:, :, :] = ogather[:, :].astype(jnp.float32).reshape(B, SQ, D)

        for rdma in sends:
            rdma.wait_send()

    return pl.pallas_call(
        body,
        out_shape=jax.ShapeDtypeStruct((B, SQ, D), jnp.float32),
        in_specs=[pl.BlockSpec(memory_space=pltpu.VMEM)] * 5,
        out_specs=pl.BlockSpec(memory_space=pltpu.VMEM),
        scratch_shapes=[
            pltpu.VMEM((QR, PW), jnp.bfloat16),
            pltpu.VMEM((QR, PW), jnp.bfloat16),
            pltpu.VMEM((QR, D), jnp.bfloat16),
            pltpu.SemaphoreType.DMA((N_DEV - 1,)),
            pltpu.SemaphoreType.DMA((N_DEV,)),
            pltpu.SemaphoreType.DMA((N_DEV - 1,)),
            pltpu.SemaphoreType.DMA((N_DEV,)),
        ],
        compiler_params=pltpu.CompilerParams(collective_id=0),
    )(x, Wq, Wo, K_ext, V_ext)
